# Initial kernel scaffold; baseline (speedup 1.0000x reference)
import jax
import jax.numpy as jnp
from jax.experimental import pallas as pl


def kernel(x, ln1_w, w_qkv, w_attn_out, ln2_w, w_router, w_fc, w_proj):
    raise NotImplementedError("write your pallas kernel here")


# speedup vs baseline: 1.0631x; 1.0631x over previous
import jax
import jax.numpy as jnp
from jax.experimental import pallas as pl
from jax.experimental.pallas import tpu as pltpu

SEQ = 2048
D = 1024
NH = 16
DH = 64
NE = 8
TOPK = 2
DFF = 4096

ROW_T = 256
TR = 256
TMAX = 24
R_PAD = TMAX * TR
NPAIR = SEQ * TOPK


def _layer_norm_x(x, w):
    mu = jnp.mean(x, axis=-1, keepdims=True)
    var = jnp.mean((x - mu) ** 2, axis=-1, keepdims=True)
    return (x - mu) / jnp.sqrt(var + 1e-5) * w


def _attention_x(x, w_qkv, w_out):
    b, s, d = x.shape
    dh = d // NH
    qkv = x @ w_qkv
    q, k, v = jnp.split(qkv, 3, axis=-1)
    q = q.reshape(b, s, NH, dh).transpose(0, 2, 1, 3)
    k = k.reshape(b, s, NH, dh).transpose(0, 2, 1, 3)
    v = v.reshape(b, s, NH, dh).transpose(0, 2, 1, 3)
    att = jnp.einsum("bhqd,bhkd->bhqk", q, k) / jnp.sqrt(jnp.asarray(dh, x.dtype))
    causal = jnp.tril(jnp.ones((s, s), dtype=bool))
    att = jnp.where(causal[None, None, :, :], att, jnp.finfo(x.dtype).min)
    att = jax.nn.softmax(att, axis=-1)
    y = jnp.einsum("bhqk,bhkd->bhqd", att, v)
    y = y.transpose(0, 2, 1, 3).reshape(b, s, d)
    return y @ w_out


def _router_body(lg_ref, pv_ref, pi_ref):
    logits = lg_ref[...]
    lm = jnp.max(logits, axis=-1, keepdims=True)
    ex = jnp.exp(logits - lm)
    probs = ex / jnp.sum(ex, axis=-1, keepdims=True)
    lane = jax.lax.broadcasted_iota(jnp.int32, (ROW_T, NE), 1)
    m1 = jnp.max(logits, axis=-1, keepdims=True)
    i1 = jnp.min(jnp.where(logits >= m1, lane, NE), axis=-1, keepdims=True)
    p1 = jnp.max(probs, axis=-1, keepdims=True)
    masked = jnp.where(lane == i1, -jnp.inf, logits)
    m2 = jnp.max(masked, axis=-1, keepdims=True)
    i2 = jnp.min(jnp.where(masked >= m2, lane, NE), axis=-1, keepdims=True)
    p2 = jnp.max(jnp.where(lane == i2, probs, -1.0), axis=-1, keepdims=True)
    tot = p1 + p2
    pv_ref[...] = jnp.concatenate([p1 / tot, p2 / tot], axis=-1)
    pi_ref[...] = jnp.concatenate([i1, i2], axis=-1)


def _router(logits):
    return pl.pallas_call(
        _router_body,
        grid=(SEQ // ROW_T,),
        in_specs=[pl.BlockSpec((ROW_T, NE), lambda i: (i, 0))],
        out_specs=[
            pl.BlockSpec((ROW_T, TOPK), lambda i: (i, 0)),
            pl.BlockSpec((ROW_T, TOPK), lambda i: (i, 0)),
        ],
        out_shape=[
            jax.ShapeDtypeStruct((SEQ, TOPK), jnp.float32),
            jax.ShapeDtypeStruct((SEQ, TOPK), jnp.int32),
        ],
    )(logits)


def _moe_body(gid_ref, xs_ref, wfc_ref, wpj_ref, wrow_ref, h_ref):
    a = xs_ref[...].astype(jnp.bfloat16)
    mid = jax.nn.gelu(jnp.dot(a, wfc_ref[0], preferred_element_type=jnp.float32))
    out = jnp.dot(mid.astype(jnp.bfloat16), wpj_ref[0],
                  preferred_element_type=jnp.float32)
    h_ref[...] = out * wrow_ref[:, :1]


def _moe_grouped(gid, xs, w_fc, w_proj, wrow2):
    grid_spec = pltpu.PrefetchScalarGridSpec(
        num_scalar_prefetch=1,
        grid=(TMAX,),
        in_specs=[
            pl.BlockSpec((TR, D), lambda t, g: (t, 0)),
            pl.BlockSpec((1, D, DFF), lambda t, g: (g[t], 0, 0)),
            pl.BlockSpec((1, DFF, D), lambda t, g: (g[t], 0, 0)),
            pl.BlockSpec((TR, 128), lambda t, g: (t, 0)),
        ],
        out_specs=pl.BlockSpec((TR, D), lambda t, g: (t, 0)),
    )
    return pl.pallas_call(
        _moe_body,
        grid_spec=grid_spec,
        out_shape=jax.ShapeDtypeStruct((R_PAD, D), jnp.float32),
    )(gid, xs, w_fc, w_proj, wrow2)


def _combine_body(xl_ref, hc_ref, o_ref):
    o_ref[...] = xl_ref[...] + (hc_ref[:, :D] + hc_ref[:, D:])


def _combine(xl, hcat):
    return pl.pallas_call(
        _combine_body,
        grid=(SEQ // ROW_T,),
        in_specs=[
            pl.BlockSpec((ROW_T, D), lambda i: (i, 0)),
            pl.BlockSpec((ROW_T, 2 * D), lambda i: (i, 0)),
        ],
        out_specs=pl.BlockSpec((ROW_T, D), lambda i: (i, 0)),
        out_shape=jax.ShapeDtypeStruct((SEQ, D), jnp.float32),
    )(xl, hcat)


def _routing_tables(pi, pv):
    e_flat = pi.reshape(-1)
    p_flat = pv.reshape(-1)
    counts = jnp.bincount(e_flat, length=NE)
    tiles_e = (counts + TR - 1) // TR
    padded = tiles_e * TR
    pstart = jnp.concatenate([jnp.zeros(1, jnp.int32),
                              jnp.cumsum(padded)[:-1].astype(jnp.int32)])
    start = jnp.concatenate([jnp.zeros(1, jnp.int32),
                             jnp.cumsum(counts)[:-1].astype(jnp.int32)])
    order = jnp.argsort(e_flat, stable=True)
    e_sorted = e_flat[order]
    rank = jnp.arange(NPAIR, dtype=jnp.int32) - start[e_sorted]
    prow = pstart[e_sorted] + rank
    row_src = jnp.zeros((R_PAD,), jnp.int32).at[prow].set(
        (order // TOPK).astype(jnp.int32))
    w_row = jnp.zeros((R_PAD,), jnp.float32).at[prow].set(p_flat[order])
    pos_flat = jnp.zeros((NPAIR,), jnp.int32).at[order].set(prow)
    tile_cum = jnp.cumsum(tiles_e)
    n_active = tile_cum[-1]
    gid = jnp.searchsorted(tile_cum, jnp.arange(TMAX, dtype=jnp.int32),
                           side="right").astype(jnp.int32)
    last_e = jnp.max(jnp.where(tiles_e > 0, jnp.arange(NE, dtype=jnp.int32), 0))
    gid = jnp.where(jnp.arange(TMAX) < n_active, jnp.minimum(gid, NE - 1), last_e)
    return gid, row_src, w_row, pos_flat


def kernel(x, ln1_w, w_qkv, w_attn_out, ln2_w, w_router, w_fc, w_proj):
    x1 = x + _attention_x(_layer_norm_x(x, ln1_w), w_qkv, w_attn_out)
    xl3 = _layer_norm_x(x1, ln2_w)
    logits = (xl3 @ w_router).reshape(SEQ, NE)
    xl = xl3.reshape(SEQ, D)

    pv, pi = _router(logits)
    gid, row_src, w_row, pos_flat = _routing_tables(pi, pv)
    xs = xl[row_src]
    wrow2 = jnp.broadcast_to(w_row[:, None], (R_PAD, 128))
    h = _moe_grouped(gid, xs, w_fc.astype(jnp.bfloat16),
                     w_proj.astype(jnp.bfloat16), wrow2)
    h_tm = h[pos_flat]
    hcat = h_tm.reshape(SEQ, 2 * D)
    out = _combine(xl, hcat)
    return out.reshape(1, SEQ, D)


# speedup vs baseline: 1.1502x; 1.0819x over previous
import jax
import jax.numpy as jnp
from jax.experimental import pallas as pl
from jax.experimental.pallas import tpu as pltpu

SEQ = 2048
D = 1024
NH = 16
DH = 64
NE = 8
TOPK = 2
DFF = 4096

ROW_T = 256
TR = 256
TMAX = 24
R_PAD = TMAX * TR
NPAIR = SEQ * TOPK
PR = 32
PC = 128


def _layer_norm_x(x, w):
    mu = jnp.mean(x, axis=-1, keepdims=True)
    var = jnp.mean((x - mu) ** 2, axis=-1, keepdims=True)
    return (x - mu) / jnp.sqrt(var + 1e-5) * w


def _attention_x(x, w_qkv, w_out):
    b, s, d = x.shape
    dh = d // NH
    qkv = x @ w_qkv
    q, k, v = jnp.split(qkv, 3, axis=-1)
    q = q.reshape(b, s, NH, dh).transpose(0, 2, 1, 3)
    k = k.reshape(b, s, NH, dh).transpose(0, 2, 1, 3)
    v = v.reshape(b, s, NH, dh).transpose(0, 2, 1, 3)
    att = jnp.einsum("bhqd,bhkd->bhqk", q, k) / jnp.sqrt(jnp.asarray(dh, x.dtype))
    causal = jnp.tril(jnp.ones((s, s), dtype=bool))
    att = jnp.where(causal[None, None, :, :], att, jnp.finfo(x.dtype).min)
    att = jax.nn.softmax(att, axis=-1)
    y = jnp.einsum("bhqk,bhkd->bhqd", att, v)
    y = y.transpose(0, 2, 1, 3).reshape(b, s, d)
    return y @ w_out


def _router_body(lg_ref, pv_ref, pi_ref):
    logits = lg_ref[...]
    lm = jnp.max(logits, axis=-1, keepdims=True)
    ex = jnp.exp(logits - lm)
    probs = ex / jnp.sum(ex, axis=-1, keepdims=True)
    lane = jax.lax.broadcasted_iota(jnp.int32, (ROW_T, NE), 1)
    m1 = jnp.max(logits, axis=-1, keepdims=True)
    i1 = jnp.min(jnp.where(logits >= m1, lane, NE), axis=-1, keepdims=True)
    p1 = jnp.max(probs, axis=-1, keepdims=True)
    masked = jnp.where(lane == i1, -jnp.inf, logits)
    m2 = jnp.max(masked, axis=-1, keepdims=True)
    i2 = jnp.min(jnp.where(masked >= m2, lane, NE), axis=-1, keepdims=True)
    p2 = jnp.max(jnp.where(lane == i2, probs, -1.0), axis=-1, keepdims=True)
    tot = p1 + p2
    pv_ref[...] = jnp.concatenate([p1 / tot, p2 / tot], axis=-1)
    pi_ref[...] = jnp.concatenate([i1, i2], axis=-1)


def _router(logits):
    return pl.pallas_call(
        _router_body,
        grid=(SEQ // ROW_T,),
        in_specs=[pl.BlockSpec((ROW_T, NE), lambda i: (i, 0))],
        out_specs=[
            pl.BlockSpec((ROW_T, TOPK), lambda i: (i, 0)),
            pl.BlockSpec((ROW_T, TOPK), lambda i: (i, 0)),
        ],
        out_shape=[
            jax.ShapeDtypeStruct((SEQ, TOPK), jnp.float32),
            jax.ShapeDtypeStruct((SEQ, TOPK), jnp.int32),
        ],
    )(logits)


def _tables_body(e_ref, prow_ref, gid_ref):
    e2d = e_ref[...]
    f32 = jnp.float32
    tri_r = jnp.tril(jnp.ones((PR, PR), f32), -1)
    tri_c = jnp.tril(jnp.ones((PC, PC), f32), -1).T
    counts = []
    masks = []
    for e in range(NE):
        m = (e2d == e).astype(f32)
        masks.append(m)
        counts.append(jnp.sum(m))
    pstarts = []
    acc = jnp.float32(0.0)
    tiles = []
    for e in range(NE):
        pstarts.append(acc)
        te = jnp.ceil(counts[e] / TR)
        tiles.append(te)
        acc = acc + te * TR
    prow = jnp.zeros((PR, PC), f32)
    for e in range(NE):
        m = masks[e]
        lane_excl = jax.lax.dot_general(
            m, tri_c, (((1,), (0,)), ((), ())), preferred_element_type=f32)
        row_tot = jnp.sum(m, axis=1, keepdims=True)
        row_excl = jax.lax.dot_general(
            tri_r, row_tot, (((1,), (0,)), ((), ())),
            preferred_element_type=f32)
        rank = lane_excl + row_excl
        prow = prow + m * (pstarts[e] + rank)
    prow_ref[...] = prow.astype(jnp.int32)

    t = jax.lax.broadcasted_iota(jnp.int32, (8, 128), 1).astype(f32)
    gid = jnp.zeros((8, 128), f32)
    cum = jnp.float32(0.0)
    n_active = jnp.float32(0.0)
    last_e = jnp.float32(0.0)
    for e in range(NE):
        lo = cum
        cum = cum + tiles[e]
        gid = gid + jnp.where((t >= lo) & (t < cum), jnp.float32(e), 0.0)
        n_active = cum
        last_e = jnp.where(tiles[e] > 0, jnp.float32(e), last_e)
    gid = jnp.where(t < n_active, gid, last_e)
    gid_ref[...] = gid.astype(jnp.int32)


def _tables(e2d):
    return pl.pallas_call(
        _tables_body,
        in_specs=[pl.BlockSpec((PR, PC), lambda: (0, 0))],
        out_specs=[
            pl.BlockSpec((PR, PC), lambda: (0, 0)),
            pl.BlockSpec((8, 128), lambda: (0, 0)),
        ],
        out_shape=[
            jax.ShapeDtypeStruct((PR, PC), jnp.int32),
            jax.ShapeDtypeStruct((8, 128), jnp.int32),
        ],
    )(e2d)


def _moe_body(gid_ref, xs_ref, wfc_ref, wpj_ref, h_ref):
    a = xs_ref[...].astype(jnp.bfloat16)
    mid = jax.nn.gelu(jnp.dot(a, wfc_ref[0], preferred_element_type=jnp.float32))
    h_ref[...] = jnp.dot(mid.astype(jnp.bfloat16), wpj_ref[0],
                         preferred_element_type=jnp.float32)


def _moe_grouped(gid, xs, w_fc, w_proj):
    grid_spec = pltpu.PrefetchScalarGridSpec(
        num_scalar_prefetch=1,
        grid=(TMAX,),
        in_specs=[
            pl.BlockSpec((TR, D), lambda t, g: (t, 0)),
            pl.BlockSpec((1, D, DFF), lambda t, g: (g[t], 0, 0)),
            pl.BlockSpec((1, DFF, D), lambda t, g: (g[t], 0, 0)),
        ],
        out_specs=pl.BlockSpec((TR, D), lambda t, g: (t, 0)),
    )
    return pl.pallas_call(
        _moe_body,
        grid_spec=grid_spec,
        out_shape=jax.ShapeDtypeStruct((R_PAD, D), jnp.float32),
    )(gid, xs, w_fc, w_proj)


def _combine_body(xl_ref, hc_ref, pv_ref, o_ref):
    pv = pv_ref[...]
    o_ref[...] = (xl_ref[...] + pv[:, 0:1] * hc_ref[:, :D]
                  + pv[:, 1:2] * hc_ref[:, D:])


def _combine(xl, hcat, pv):
    return pl.pallas_call(
        _combine_body,
        grid=(SEQ // ROW_T,),
        in_specs=[
            pl.BlockSpec((ROW_T, D), lambda i: (i, 0)),
            pl.BlockSpec((ROW_T, 2 * D), lambda i: (i, 0)),
            pl.BlockSpec((ROW_T, TOPK), lambda i: (i, 0)),
        ],
        out_specs=pl.BlockSpec((ROW_T, D), lambda i: (i, 0)),
        out_shape=jax.ShapeDtypeStruct((SEQ, D), jnp.float32),
    )(xl, hcat, pv)


def kernel(x, ln1_w, w_qkv, w_attn_out, ln2_w, w_router, w_fc, w_proj):
    x1 = x + _attention_x(_layer_norm_x(x, ln1_w), w_qkv, w_attn_out)
    xl3 = _layer_norm_x(x1, ln2_w)
    logits = (xl3 @ w_router).reshape(SEQ, NE)
    xl = xl3.reshape(SEQ, D)

    pv, pi = _router(logits)
    prow2d, gidpad = _tables(pi.reshape(PR, PC))
    prow = prow2d.reshape(NPAIR)
    gid = gidpad.reshape(-1)[:TMAX]

    xpairs = jnp.repeat(xl, TOPK, axis=0)
    xs = jnp.zeros((R_PAD, D), jnp.float32).at[prow].set(xpairs)
    h = _moe_grouped(gid, xs, w_fc.astype(jnp.bfloat16),
                     w_proj.astype(jnp.bfloat16))
    h_tm = h[prow]
    hcat = h_tm.reshape(SEQ, 2 * D)
    out = _combine(xl, hcat, pv)
    return out.reshape(1, SEQ, D)


# speedup vs baseline: 1.1593x; 1.0079x over previous
import functools
import jax
import jax.numpy as jnp
from jax import lax
from jax.experimental import pallas as pl
from jax.experimental.pallas import tpu as pltpu
from jax.experimental.pallas import tpu_sc as plsc

SEQ = 2048
D = 1024
NH = 16
DH = 64
NE = 8
TOPK = 2
DFF = 4096

ROW_T = 256
TR = 256
TMAX = 24
R_PAD = TMAX * TR
NPAIR = SEQ * TOPK
PR = 32
PC = 128


def _layer_norm_x(x, w):
    mu = jnp.mean(x, axis=-1, keepdims=True)
    var = jnp.mean((x - mu) ** 2, axis=-1, keepdims=True)
    return (x - mu) / jnp.sqrt(var + 1e-5) * w


def _attention_x(x, w_qkv, w_out):
    b, s, d = x.shape
    dh = d // NH
    qkv = x @ w_qkv
    q, k, v = jnp.split(qkv, 3, axis=-1)
    q = q.reshape(b, s, NH, dh).transpose(0, 2, 1, 3)
    k = k.reshape(b, s, NH, dh).transpose(0, 2, 1, 3)
    v = v.reshape(b, s, NH, dh).transpose(0, 2, 1, 3)
    att = jnp.einsum("bhqd,bhkd->bhqk", q, k) / jnp.sqrt(jnp.asarray(dh, x.dtype))
    causal = jnp.tril(jnp.ones((s, s), dtype=bool))
    att = jnp.where(causal[None, None, :, :], att, jnp.finfo(x.dtype).min)
    att = jax.nn.softmax(att, axis=-1)
    y = jnp.einsum("bhqk,bhkd->bhqd", att, v)
    y = y.transpose(0, 2, 1, 3).reshape(b, s, d)
    return y @ w_out


def _router_body(lg_ref, pv_ref, pi_ref):
    logits = lg_ref[...]
    lm = jnp.max(logits, axis=-1, keepdims=True)
    ex = jnp.exp(logits - lm)
    probs = ex / jnp.sum(ex, axis=-1, keepdims=True)
    lane = jax.lax.broadcasted_iota(jnp.int32, (ROW_T, NE), 1)
    m1 = jnp.max(logits, axis=-1, keepdims=True)
    i1 = jnp.min(jnp.where(logits >= m1, lane, NE), axis=-1, keepdims=True)
    p1 = jnp.max(probs, axis=-1, keepdims=True)
    masked = jnp.where(lane == i1, -jnp.inf, logits)
    m2 = jnp.max(masked, axis=-1, keepdims=True)
    i2 = jnp.min(jnp.where(masked >= m2, lane, NE), axis=-1, keepdims=True)
    p2 = jnp.max(jnp.where(lane == i2, probs, -1.0), axis=-1, keepdims=True)
    tot = p1 + p2
    pv_ref[...] = jnp.concatenate([p1 / tot, p2 / tot], axis=-1)
    pi_ref[...] = jnp.concatenate([i1, i2], axis=-1)


def _router(logits):
    return pl.pallas_call(
        _router_body,
        grid=(SEQ // ROW_T,),
        in_specs=[pl.BlockSpec((ROW_T, NE), lambda i: (i, 0))],
        out_specs=[
            pl.BlockSpec((ROW_T, TOPK), lambda i: (i, 0)),
            pl.BlockSpec((ROW_T, TOPK), lambda i: (i, 0)),
        ],
        out_shape=[
            jax.ShapeDtypeStruct((SEQ, TOPK), jnp.float32),
            jax.ShapeDtypeStruct((SEQ, TOPK), jnp.int32),
        ],
    )(logits)


def _tables_body(e_ref, prow_ref, gid_ref):
    e2d = e_ref[...]
    f32 = jnp.float32
    tri_r = jnp.tril(jnp.ones((PR, PR), f32), -1)
    tri_c = jnp.tril(jnp.ones((PC, PC), f32), -1).T
    counts = []
    masks = []
    for e in range(NE):
        m = (e2d == e).astype(f32)
        masks.append(m)
        counts.append(jnp.sum(m))
    pstarts = []
    acc = jnp.float32(0.0)
    tiles = []
    for e in range(NE):
        pstarts.append(acc)
        te = jnp.ceil(counts[e] / TR)
        tiles.append(te)
        acc = acc + te * TR
    prow = jnp.zeros((PR, PC), f32)
    for e in range(NE):
        m = masks[e]
        lane_excl = jax.lax.dot_general(
            m, tri_c, (((1,), (0,)), ((), ())), preferred_element_type=f32)
        row_tot = jnp.sum(m, axis=1, keepdims=True)
        row_excl = jax.lax.dot_general(
            tri_r, row_tot, (((1,), (0,)), ((), ())),
            preferred_element_type=f32)
        rank = lane_excl + row_excl
        prow = prow + m * (pstarts[e] + rank)
    prow_ref[...] = prow.astype(jnp.int32)

    t = jax.lax.broadcasted_iota(jnp.int32, (8, 128), 1).astype(f32)
    gid = jnp.zeros((8, 128), f32)
    cum = jnp.float32(0.0)
    n_active = jnp.float32(0.0)
    last_e = jnp.float32(0.0)
    for e in range(NE):
        lo = cum
        cum = cum + tiles[e]
        gid = gid + jnp.where((t >= lo) & (t < cum), jnp.float32(e), 0.0)
        n_active = cum
        last_e = jnp.where(tiles[e] > 0, jnp.float32(e), last_e)
    gid = jnp.where(t < n_active, gid, last_e)
    gid_ref[...] = gid.astype(jnp.int32)


def _tables(e2d):
    return pl.pallas_call(
        _tables_body,
        in_specs=[pl.BlockSpec((PR, PC), lambda: (0, 0))],
        out_specs=[
            pl.BlockSpec((PR, PC), lambda: (0, 0)),
            pl.BlockSpec((8, 128), lambda: (0, 0)),
        ],
        out_shape=[
            jax.ShapeDtypeStruct((PR, PC), jnp.int32),
            jax.ShapeDtypeStruct((8, 128), jnp.int32),
        ],
    )(e2d)


def _moe_body(gid_ref, xs_ref, wfc_ref, wpj_ref, h_ref):
    a = xs_ref[...].astype(jnp.bfloat16)
    mid = jax.nn.gelu(jnp.dot(a, wfc_ref[0], preferred_element_type=jnp.float32))
    h_ref[...] = jnp.dot(mid.astype(jnp.bfloat16), wpj_ref[0],
                         preferred_element_type=jnp.float32)


def _moe_grouped(gid, xs, w_fc, w_proj):
    grid_spec = pltpu.PrefetchScalarGridSpec(
        num_scalar_prefetch=1,
        grid=(TMAX,),
        in_specs=[
            pl.BlockSpec((TR, D), lambda t, g: (t, 0)),
            pl.BlockSpec((1, D, DFF), lambda t, g: (g[t], 0, 0)),
            pl.BlockSpec((1, DFF, D), lambda t, g: (g[t], 0, 0)),
        ],
        out_specs=pl.BlockSpec((TR, D), lambda t, g: (t, 0)),
    )
    return pl.pallas_call(
        _moe_body,
        grid_spec=grid_spec,
        out_shape=jax.ShapeDtypeStruct((R_PAD, D), jnp.float32),
    )(gid, xs, w_fc, w_proj)


def _sc_gather(table, idx):
    NW = 32
    b_per_w = NPAIR // NW
    CH = 64

    mesh = plsc.VectorSubcoreMesh(core_axis_name="c", subcore_axis_name="s")

    @functools.partial(
        pl.kernel, mesh=mesh,
        out_type=jax.ShapeDtypeStruct((NPAIR, D), jnp.float32),
        scratch_types=[
            pltpu.VMEM((CH,), jnp.int32),
            pltpu.VMEM((CH, D), jnp.float32),
            pltpu.SemaphoreType.DMA,
        ],
    )
    def k(table_hbm, idx_hbm, out_hbm, idx_v, rows_v, sem):
        wid = lax.axis_index("s") * 2 + lax.axis_index("c")
        base = wid * b_per_w
        for c in range(b_per_w // CH):
            off = base + c * CH
            pltpu.sync_copy(idx_hbm.at[pl.ds(off, CH)], idx_v)
            pltpu.async_copy(table_hbm.at[idx_v], rows_v, sem).wait()
            pltpu.sync_copy(rows_v, out_hbm.at[pl.ds(off, CH)])

    return k(table, idx)


def _combine_body(xl_ref, hc_ref, pv_ref, o_ref):
    pv = pv_ref[...]
    o_ref[...] = (xl_ref[...] + pv[:, 0:1] * hc_ref[:, :D]
                  + pv[:, 1:2] * hc_ref[:, D:])


def _combine(xl, hcat, pv):
    return pl.pallas_call(
        _combine_body,
        grid=(SEQ // ROW_T,),
        in_specs=[
            pl.BlockSpec((ROW_T, D), lambda i: (i, 0)),
            pl.BlockSpec((ROW_T, 2 * D), lambda i: (i, 0)),
            pl.BlockSpec((ROW_T, TOPK), lambda i: (i, 0)),
        ],
        out_specs=pl.BlockSpec((ROW_T, D), lambda i: (i, 0)),
        out_shape=jax.ShapeDtypeStruct((SEQ, D), jnp.float32),
    )(xl, hcat, pv)


def kernel(x, ln1_w, w_qkv, w_attn_out, ln2_w, w_router, w_fc, w_proj):
    x1 = x + _attention_x(_layer_norm_x(x, ln1_w), w_qkv, w_attn_out)
    xl3 = _layer_norm_x(x1, ln2_w)
    logits = (xl3 @ w_router).reshape(SEQ, NE)
    xl = xl3.reshape(SEQ, D)

    pv, pi = _router(logits)
    prow2d, gidpad = _tables(pi.reshape(PR, PC))
    prow = prow2d.reshape(NPAIR)
    gid = gidpad.reshape(-1)[:TMAX]

    prow2 = prow.reshape(SEQ, TOPK)
    xs = jnp.zeros((R_PAD, D), jnp.float32)
    xs = xs.at[prow2[:, 0]].set(xl).at[prow2[:, 1]].set(xl)
    h = _moe_grouped(gid, xs, w_fc.astype(jnp.bfloat16),
                     w_proj.astype(jnp.bfloat16))
    h_tm = _sc_gather(h, prow)
    hcat = h_tm.reshape(SEQ, 2 * D)
    out = _combine(xl, hcat, pv)
    return out.reshape(1, SEQ, D)
